# Initial kernel scaffold; baseline (speedup 1.0000x reference)
import jax
import jax.numpy as jnp
from jax.experimental import pallas as pl


def kernel(node_feats, edge_index, W_self, W_neigh, b_sage, W_dec, b_dec):
    raise NotImplementedError("write your pallas kernel here")


# speedup vs baseline: 5.7018x; 5.7018x over previous
import functools

import jax
import jax.numpy as jnp
from jax import lax
from jax.experimental import pallas as pl
from jax.experimental.pallas import tpu as pltpu
from jax.experimental.pallas import tpu_sc as plsc

N = 10000
F = 128
NP = 10112
NC, NS = 2, 16
NW = NC * NS
CHUNK = 128
CPT = 79
EPT = CPT * CHUNK
ROWS_PT = NP // NS


def _sc_body(node_hbm, src_hbm, dst_hbm, agg_out, deg_out,
             src_v, dst_v, rows_v, deg_local, agg_sh, sem):
    c = lax.axis_index("c")
    s = lax.axis_index("s")

    pltpu.sync_copy(src_hbm.at[c, s], src_v)
    pltpu.sync_copy(dst_hbm.at[c, s], dst_v)

    z16 = jnp.zeros((16,), jnp.float32)

    def _zero_rows(i, carry):
        rows_v[i // 8, pl.ds((i % 8) * 16, 16)] = z16
        return carry
    lax.fori_loop(0, 128 * 8, _zero_rows, 0)

    def _zero_deg(i, carry):
        deg_local[pl.ds(i * 16, 16)] = z16
        return carry
    lax.fori_loop(0, NP // 16, _zero_deg, 0)

    base = s * ROWS_PT
    for k in range(4):
        pltpu.sync_copy(rows_v, agg_sh.at[pl.ds(base + k * 128, 128)])
    pltpu.sync_copy(rows_v.at[pl.ds(0, ROWS_PT - 512)],
                    agg_sh.at[pl.ds(base + 512, ROWS_PT - 512)])
    plsc.subcore_barrier()

    ones16 = jnp.ones((16,), jnp.float32)

    def _chunk(j, carry):
        pltpu.async_copy(node_hbm.at[src_v.at[j]], rows_v, sem).wait()
        pltpu.sync_copy(rows_v, agg_sh.at[dst_v.at[j]], add=True)

        def _hist(k, inner):
            d16 = dst_v[j, pl.ds(k * 16, 16)]
            plsc.addupdate_scatter(deg_local, [d16], ones16)
            return inner
        lax.fori_loop(0, CHUNK // 16, _hist, 0)
        return carry
    lax.fori_loop(0, CPT, _chunk, 0)

    plsc.subcore_barrier()
    pltpu.sync_copy(agg_sh.at[pl.ds(base, ROWS_PT)],
                    agg_out.at[c, pl.ds(base, ROWS_PT)])
    pltpu.sync_copy(deg_local, deg_out.at[c * NS + s])


_sc_agg = functools.partial(
    pl.kernel,
    out_type=(
        jax.ShapeDtypeStruct((NC, NP, F), jnp.float32),
        jax.ShapeDtypeStruct((NW, NP), jnp.float32),
    ),
    mesh=plsc.VectorSubcoreMesh(core_axis_name="c", subcore_axis_name="s"),
    compiler_params=pltpu.CompilerParams(needs_layout_passes=False),
    scratch_types=[
        pltpu.VMEM((CPT, CHUNK), jnp.int32),
        pltpu.VMEM((CPT, CHUNK), jnp.int32),
        pltpu.VMEM((CHUNK, F), jnp.float32),
        pltpu.VMEM((NP,), jnp.float32),
        pltpu.VMEM_SHARED((NP, F), jnp.float32),
        pltpu.SemaphoreType.DMA,
    ],
)(_sc_body)


def _tc_body(node_ref, agg0_ref, agg1_ref, degp_ref,
             ws_ref, wn_ref, bs_ref, wd_ref, bd_ref, z_ref, rec_ref):
    deg = jnp.sum(degp_ref[...], axis=1)
    agg = agg0_ref[0] + agg1_ref[0]
    hn = jnp.where(deg[:, None] > 0.0,
                   agg / jnp.maximum(deg, 1.0)[:, None], 0.0)
    h = (jnp.dot(node_ref[...], ws_ref[...], preferred_element_type=jnp.float32)
         + jnp.dot(hn, wn_ref[...], preferred_element_type=jnp.float32)
         + bs_ref[...])
    z = jnp.maximum(h, 0.0)
    z_ref[...] = z
    rec_ref[...] = (jnp.dot(z, wd_ref[...], preferred_element_type=jnp.float32)
                    + bd_ref[...])


NB = 1000


def _tc_call(node_feats, agg_p, deg_p, W_self, W_neigh, b_sage, W_dec, b_dec):
    grid = N // NB
    row_spec = pl.BlockSpec((NB, F), lambda i: (i, 0))
    full_spec = pl.BlockSpec((F, F), lambda i: (0, 0))
    bias_spec = pl.BlockSpec((1, F), lambda i: (0, 0))
    return pl.pallas_call(
        _tc_body,
        grid=(grid,),
        in_specs=[
            row_spec,
            pl.BlockSpec((1, NB, F), lambda i: (0, i, 0)),
            pl.BlockSpec((1, NB, F), lambda i: (1, i, 0)),
            pl.BlockSpec((NB, NW), lambda i: (i, 0)),
            full_spec, full_spec, bias_spec, full_spec, bias_spec,
        ],
        out_specs=[row_spec, row_spec],
        out_shape=[
            jax.ShapeDtypeStruct((N, F), jnp.float32),
            jax.ShapeDtypeStruct((N, F), jnp.float32),
        ],
    )(node_feats, agg_p, agg_p, deg_p,
      W_self, W_neigh, b_sage.reshape(1, F), W_dec, b_dec.reshape(1, F))


def kernel(node_feats, edge_index, W_self, W_neigh, b_sage, W_dec, b_dec):
    src = edge_index[0]
    dst = edge_index[1]
    e = src.shape[0]
    pad = NW * EPT - e
    src_p = jnp.concatenate(
        [src, jnp.zeros((pad,), jnp.int32)]).reshape(NC, NS, CPT, CHUNK)
    dst_p = jnp.concatenate(
        [dst, jnp.full((pad,), N, jnp.int32)]).reshape(NC, NS, CPT, CHUNK)
    agg_p, deg_p = _sc_agg(node_feats, src_p, dst_p)
    z, recon = _tc_call(node_feats, agg_p, deg_p[:, :N].T,
                        W_self, W_neigh, b_sage, W_dec, b_dec)
    return (z, recon)


# speedup vs baseline: 5.7748x; 1.0128x over previous
import functools

import jax
import jax.numpy as jnp
from jax import lax
from jax.experimental import pallas as pl
from jax.experimental.pallas import tpu as pltpu
from jax.experimental.pallas import tpu_sc as plsc

N = 10000
F = 128
NP = 10112
NC, NS = 2, 16
NW = NC * NS
CHUNK = 128
CPT = 79
EPT = CPT * CHUNK
ROWS_PT = NP // NS


def _sc_body(node_hbm, src_hbm, dst_hbm, agg_out, deg_out,
             src_v, dst_v, rows_v, deg_local, agg_sh, sem):
    c = lax.axis_index("c")
    s = lax.axis_index("s")

    pltpu.sync_copy(src_hbm.at[c, s], src_v)
    pltpu.sync_copy(dst_hbm.at[c, s], dst_v)

    z16 = jnp.zeros((16,), jnp.float32)

    def _zero_rows(i, carry):
        rows_v[i // 8, pl.ds((i % 8) * 16, 16)] = z16
        return carry
    lax.fori_loop(0, 128 * 8, _zero_rows, 0)

    def _zero_deg(i, carry):
        deg_local[pl.ds(i * 16, 16)] = z16
        return carry
    lax.fori_loop(0, NP // 16, _zero_deg, 0)

    base = s * ROWS_PT
    for k in range(4):
        pltpu.sync_copy(rows_v, agg_sh.at[pl.ds(base + k * 128, 128)])
    pltpu.sync_copy(rows_v.at[pl.ds(0, ROWS_PT - 512)],
                    agg_sh.at[pl.ds(base + 512, ROWS_PT - 512)])
    plsc.subcore_barrier()

    ones16 = jnp.ones((16,), jnp.float32)

    def _chunk(j, carry):
        pltpu.async_copy(node_hbm.at[src_v.at[j]], rows_v, sem).wait()
        pltpu.sync_copy(rows_v, agg_sh.at[dst_v.at[j]], add=True)

        def _hist(k, inner):
            d16 = dst_v[j, pl.ds(k * 16, 16)]
            plsc.addupdate_scatter(deg_local, [d16], ones16)
            return inner
        lax.fori_loop(0, CHUNK // 16, _hist, 0)
        return carry
    lax.fori_loop(0, CPT, _chunk, 0)

    plsc.subcore_barrier()
    pltpu.sync_copy(agg_sh.at[pl.ds(base, ROWS_PT)],
                    agg_out.at[c, pl.ds(base, ROWS_PT)])
    pltpu.sync_copy(deg_local, deg_out.at[c * NS + s])


_sc_agg = functools.partial(
    pl.kernel,
    out_type=(
        jax.ShapeDtypeStruct((NC, NP, F), jnp.float32),
        jax.ShapeDtypeStruct((NW, NP), jnp.float32),
    ),
    mesh=plsc.VectorSubcoreMesh(core_axis_name="c", subcore_axis_name="s"),
    compiler_params=pltpu.CompilerParams(needs_layout_passes=False),
    scratch_types=[
        pltpu.VMEM((CPT, CHUNK), jnp.int32),
        pltpu.VMEM((CPT, CHUNK), jnp.int32),
        pltpu.VMEM((CHUNK, F), jnp.float32),
        pltpu.VMEM((NP,), jnp.float32),
        pltpu.VMEM_SHARED((NP, F), jnp.float32),
        pltpu.SemaphoreType.DMA,
    ],
)(_sc_body)


def _tc_body(node_ref, agg0_ref, agg1_ref, degp_ref,
             ws_ref, wn_ref, bs_ref, wd_ref, bd_ref, z_ref, rec_ref):
    deg = jnp.sum(degp_ref[...], axis=1)
    agg = agg0_ref[0] + agg1_ref[0]
    hn = jnp.where(deg[:, None] > 0.0,
                   agg / jnp.maximum(deg, 1.0)[:, None], 0.0)
    h = (jnp.dot(node_ref[...], ws_ref[...], preferred_element_type=jnp.float32)
         + jnp.dot(hn, wn_ref[...], preferred_element_type=jnp.float32)
         + bs_ref[...])
    z = jnp.maximum(h, 0.0)
    z_ref[...] = z
    rec_ref[...] = (jnp.dot(z, wd_ref[...], preferred_element_type=jnp.float32)
                    + bd_ref[...])


NB = 1000


def _tc_call(node_feats, agg_p, deg_p, W_self, W_neigh, b_sage, W_dec, b_dec):
    grid = N // NB
    row_spec = pl.BlockSpec((NB, F), lambda i: (i, 0))
    full_spec = pl.BlockSpec((F, F), lambda i: (0, 0))
    bias_spec = pl.BlockSpec((1, F), lambda i: (0, 0))
    return pl.pallas_call(
        _tc_body,
        grid=(grid,),
        in_specs=[
            row_spec,
            pl.BlockSpec((1, NB, F), lambda i: (0, i, 0)),
            pl.BlockSpec((1, NB, F), lambda i: (1, i, 0)),
            pl.BlockSpec((NB, NW), lambda i: (i, 0)),
            full_spec, full_spec, bias_spec, full_spec, bias_spec,
        ],
        out_specs=[row_spec, row_spec],
        out_shape=[
            jax.ShapeDtypeStruct((N, F), jnp.float32),
            jax.ShapeDtypeStruct((N, F), jnp.float32),
        ],
    )(node_feats, agg_p, agg_p, deg_p,
      W_self, W_neigh, b_sage.reshape(1, F), W_dec, b_dec.reshape(1, F))


def kernel(node_feats, edge_index, W_self, W_neigh, b_sage, W_dec, b_dec):
    src = edge_index[0]
    dst = edge_index[1]
    e = src.shape[0]
    pad = NW * EPT - e
    src_p = jnp.concatenate(
        [src, jnp.zeros((pad,), jnp.int32)]).reshape(NC, NS, CPT, CHUNK)
    trash = N + jnp.arange(pad, dtype=jnp.int32) % (NP - N)
    dst_p = jnp.concatenate([dst, trash]).reshape(NC, NS, CPT, CHUNK)
    agg_p, deg_p = _sc_agg(node_feats, src_p, dst_p)
    z, recon = _tc_call(node_feats, agg_p, deg_p[:, :N].T,
                        W_self, W_neigh, b_sage, W_dec, b_dec)
    return (z, recon)
